# baseline (speedup 1.0000x reference)
import functools

import jax
import jax.numpy as jnp
from jax import lax
from jax.experimental import pallas as pl
from jax.experimental.pallas import tpu as pltpu, tpu_sc as plsc

N_NODES = 10000
D_IN = 128
HIDDEN = 32
D_OUT = 16
N_EDGES = 640000

NC = 2
NS = 16
NW = NC * NS
CHUNK = 128
K_CHUNKS = 158
E_PAD = NW * K_CHUNKS * CHUNK
ACC_N = 10240
DUMP_ROW = N_NODES
STRIPE = ACC_N // NS


def _make_segsum(feat_dim):
    mesh = plsc.VectorSubcoreMesh(core_axis_name="c", subcore_axis_name="s")

    @functools.partial(
        pl.kernel,
        out_type=jax.ShapeDtypeStruct((NC, 2, ACC_N, feat_dim), jnp.float32),
        mesh=mesh,
        scratch_types=[
            pltpu.VMEM((K_CHUNKS, CHUNK), jnp.int32),
            pltpu.VMEM((K_CHUNKS, CHUNK), jnp.int32),
            [pltpu.VMEM((CHUNK, feat_dim), jnp.float32)] * 2,
            [pltpu.VMEM_SHARED((ACC_N, feat_dim), jnp.float32)] * 2,
            [pltpu.SemaphoreType.DMA] * 2,
        ],
        compiler_params=pltpu.CompilerParams(use_tc_tiling_on_sc=False),
    )
    def segsum(h_hbm, src_hbm, dst_hbm, zero_hbm, out_hbm,
               srcv, dstv, rows, acc, gsem):
        c = lax.axis_index("c")
        s = lax.axis_index("s")
        wid = s * NC + c

        pltpu.sync_copy(zero_hbm.at[pl.ds(s * STRIPE, STRIPE)],
                        acc[0].at[pl.ds(s * STRIPE, STRIPE)])
        pltpu.sync_copy(zero_hbm.at[pl.ds(s * STRIPE, STRIPE)],
                        acc[1].at[pl.ds(s * STRIPE, STRIPE)])
        pltpu.sync_copy(src_hbm.at[wid], srcv)
        pltpu.sync_copy(dst_hbm.at[wid], dstv)
        plsc.subcore_barrier()

        pltpu.async_copy(h_hbm.at[srcv.at[0]], rows[0], gsem[0])

        @pl.loop(0, K_CHUNKS, step=2)
        def _(j):
            pltpu.async_copy(h_hbm.at[srcv.at[j + 1]], rows[1], gsem[1])
            pltpu.make_async_copy(
                h_hbm.at[srcv.at[j]], rows[0], gsem[0]).wait()
            pltpu.sync_copy(rows[0], acc[0].at[dstv.at[j]], add=True)

            @pl.when(j + 2 < K_CHUNKS)
            def _():
                pltpu.async_copy(h_hbm.at[srcv.at[j + 2]], rows[0], gsem[0])

            pltpu.make_async_copy(
                h_hbm.at[srcv.at[j + 1]], rows[1], gsem[1]).wait()
            pltpu.sync_copy(rows[1], acc[1].at[dstv.at[j + 1]], add=True)

        plsc.subcore_barrier()
        pltpu.sync_copy(acc[0].at[pl.ds(s * STRIPE, STRIPE)],
                        out_hbm.at[c, 0, pl.ds(s * STRIPE, STRIPE)])
        pltpu.sync_copy(acc[1].at[pl.ds(s * STRIPE, STRIPE)],
                        out_hbm.at[c, 1, pl.ds(s * STRIPE, STRIPE)])

    return segsum


_segsum_h = _make_segsum(HIDDEN)
_segsum_z = _make_segsum(D_OUT)


def _mm0_body(x_ref, w_ref, o_ref):
    o_ref[...] = jnp.dot(x_ref[...], w_ref[...],
                         preferred_element_type=jnp.float32)


def _mm1_body(ha_ref, hb_ref, hc_ref, hd_ref, w_ref, o_ref):
    h = jnp.maximum((ha_ref[0, 0] + hb_ref[0, 0]) +
                    (hc_ref[0, 0] + hd_ref[0, 0]), 0.0)
    o_ref[...] = jnp.dot(h, w_ref[...], preferred_element_type=jnp.float32)


_PRED_BM = 256


def _pred_body(za_ref, zb_ref, zc_ref, zd_ref, z_ref, pred_ref, mat_ref):
    i = pl.program_id(0)
    zfull = (za_ref[0, 0, pl.ds(0, N_NODES), :] +
             zb_ref[0, 0, pl.ds(0, N_NODES), :] +
             zc_ref[0, 0, pl.ds(0, N_NODES), :] +
             zd_ref[0, 0, pl.ds(0, N_NODES), :])
    sl = pl.ds(i * _PRED_BM, _PRED_BM)
    zblk = (za_ref[0, 0, sl, :] + zb_ref[0, 0, sl, :] +
            zc_ref[0, 0, sl, :] + zd_ref[0, 0, sl, :])
    z_ref[...] = zblk
    mat_ref[...] = lax.dot_general(
        zblk, zfull, (((1,), (1,)), ((), ())),
        preferred_element_type=jnp.float32)
    for r in range(_PRED_BM):
        pred_ref[pl.ds(r * N_NODES, N_NODES)] = mat_ref[r, :]


def kernel(x, edge_index, W0, W1):
    src = edge_index[0].astype(jnp.int32)
    dst = edge_index[1].astype(jnp.int32)
    pad = E_PAD - N_EDGES
    src_p = jnp.concatenate([src, jnp.zeros((pad,), jnp.int32)]
                            ).reshape(NW, K_CHUNKS, CHUNK)
    dst_p = jnp.concatenate([dst, jnp.full((pad,), DUMP_ROW, jnp.int32)]
                            ).reshape(NW, K_CHUNKS, CHUNK)
    zero_h = jnp.zeros((ACC_N, HIDDEN), jnp.float32)
    zero_z = jnp.zeros((ACC_N, D_OUT), jnp.float32)

    h0 = pl.pallas_call(
        _mm0_body,
        out_shape=jax.ShapeDtypeStruct((N_NODES, HIDDEN), jnp.float32),
    )(x, W0)

    h_parts = _segsum_h(h0, src_p, dst_p, zero_h)

    z0 = pl.pallas_call(
        _mm1_body,
        grid=(1,),
        in_specs=[
            pl.BlockSpec((1, 1, N_NODES, HIDDEN), lambda i: (0, 0, 0, 0)),
            pl.BlockSpec((1, 1, N_NODES, HIDDEN), lambda i: (0, 1, 0, 0)),
            pl.BlockSpec((1, 1, N_NODES, HIDDEN), lambda i: (1, 0, 0, 0)),
            pl.BlockSpec((1, 1, N_NODES, HIDDEN), lambda i: (1, 1, 0, 0)),
            pl.BlockSpec((HIDDEN, D_OUT), lambda i: (0, 0)),
        ],
        out_specs=pl.BlockSpec((N_NODES, D_OUT), lambda i: (0, 0)),
        out_shape=jax.ShapeDtypeStruct((N_NODES, D_OUT), jnp.float32),
    )(h_parts, h_parts, h_parts, h_parts, W1)

    z_parts = _segsum_z(z0, src_p, dst_p, zero_z)

    grid = -(-N_NODES // _PRED_BM)
    z, pred = pl.pallas_call(
        _pred_body,
        grid=(grid,),
        in_specs=[
            pl.BlockSpec((1, 1, ACC_N, D_OUT), lambda i: (0, 0, 0, 0)),
            pl.BlockSpec((1, 1, ACC_N, D_OUT), lambda i: (0, 1, 0, 0)),
            pl.BlockSpec((1, 1, ACC_N, D_OUT), lambda i: (1, 0, 0, 0)),
            pl.BlockSpec((1, 1, ACC_N, D_OUT), lambda i: (1, 1, 0, 0)),
        ],
        out_specs=[
            pl.BlockSpec((_PRED_BM, D_OUT), lambda i: (i, 0)),
            pl.BlockSpec((_PRED_BM * N_NODES,), lambda i: (i,)),
        ],
        out_shape=[
            jax.ShapeDtypeStruct((N_NODES, D_OUT), jnp.float32),
            jax.ShapeDtypeStruct((N_NODES * N_NODES,), jnp.float32),
        ],
        scratch_shapes=[pltpu.VMEM((_PRED_BM, N_NODES), jnp.float32)],
    )(z_parts, z_parts, z_parts, z_parts)

    return z, pred


# speedup vs baseline: 1.0887x; 1.0887x over previous
import functools

import jax
import jax.numpy as jnp
from jax import lax
from jax.experimental import pallas as pl
from jax.experimental.pallas import tpu as pltpu, tpu_sc as plsc

N_NODES = 10000
D_IN = 128
HIDDEN = 32
D_OUT = 16
N_EDGES = 640000

NC = 2
NS = 16
NW = NC * NS
CHUNK = 128
K_CHUNKS = 158
E_PAD = NW * K_CHUNKS * CHUNK
ACC_N = 10240
DUMP_ROW = N_NODES
STRIPE = ACC_N // NS


def _make_segsum(feat_dim):
    mesh = plsc.VectorSubcoreMesh(core_axis_name="c", subcore_axis_name="s")

    @functools.partial(
        pl.kernel,
        out_type=jax.ShapeDtypeStruct((NC, ACC_N, feat_dim), jnp.float32),
        mesh=mesh,
        scratch_types=[
            pltpu.VMEM((K_CHUNKS, CHUNK), jnp.int32),
            pltpu.VMEM((K_CHUNKS, CHUNK), jnp.int32),
            [pltpu.VMEM((CHUNK, feat_dim), jnp.float32)] * 2,
            pltpu.VMEM_SHARED((ACC_N, feat_dim), jnp.float32),
            [pltpu.SemaphoreType.DMA] * 2,
        ],
        compiler_params=pltpu.CompilerParams(use_tc_tiling_on_sc=False),
    )
    def segsum(h_hbm, src_hbm, dst_hbm, zero_hbm, out_hbm,
               srcv, dstv, rows, acc, gsem):
        c = lax.axis_index("c")
        s = lax.axis_index("s")
        wid = s * NC + c

        pltpu.sync_copy(zero_hbm.at[pl.ds(s * STRIPE, STRIPE)],
                        acc.at[pl.ds(s * STRIPE, STRIPE)])
        pltpu.sync_copy(src_hbm.at[wid], srcv)
        pltpu.sync_copy(dst_hbm.at[wid], dstv)
        plsc.subcore_barrier()

        pltpu.async_copy(h_hbm.at[srcv.at[0]], rows[0], gsem[0])

        @pl.loop(0, K_CHUNKS, step=2)
        def _(j):
            pltpu.async_copy(h_hbm.at[srcv.at[j + 1]], rows[1], gsem[1])
            pltpu.make_async_copy(
                h_hbm.at[srcv.at[j]], rows[0], gsem[0]).wait()
            pltpu.sync_copy(rows[0], acc.at[dstv.at[j]], add=True)

            @pl.when(j + 2 < K_CHUNKS)
            def _():
                pltpu.async_copy(h_hbm.at[srcv.at[j + 2]], rows[0], gsem[0])

            pltpu.make_async_copy(
                h_hbm.at[srcv.at[j + 1]], rows[1], gsem[1]).wait()
            pltpu.sync_copy(rows[1], acc.at[dstv.at[j + 1]], add=True)

        plsc.subcore_barrier()
        pltpu.sync_copy(acc.at[pl.ds(s * STRIPE, STRIPE)],
                        out_hbm.at[c, pl.ds(s * STRIPE, STRIPE)])

    return segsum


_segsum_h = _make_segsum(HIDDEN)
_segsum_z = _make_segsum(D_OUT)


def _mm0_body(x_ref, w_ref, o_ref):
    o_ref[...] = jnp.dot(x_ref[...], w_ref[...],
                         preferred_element_type=jnp.float32)


def _mm1_body(ha_ref, hb_ref, w_ref, o_ref):
    h = jnp.maximum(ha_ref[0] + hb_ref[0], 0.0)
    o_ref[...] = jnp.dot(h, w_ref[...], preferred_element_type=jnp.float32)


_PRED_BM = 320


def _pred_body(za_ref, zb_ref, z_ref, pred_ref, mat_ref):
    i = pl.program_id(0)
    zfull = za_ref[0, pl.ds(0, N_NODES), :] + zb_ref[0, pl.ds(0, N_NODES), :]
    sl = pl.ds(i * _PRED_BM, _PRED_BM)
    zblk = za_ref[0, sl, :] + zb_ref[0, sl, :]
    z_ref[...] = zblk
    mat_ref[...] = lax.dot_general(
        zblk, zfull, (((1,), (1,)), ((), ())),
        preferred_element_type=jnp.float32)
    for r in range(_PRED_BM):
        pred_ref[pl.ds(r * N_NODES, N_NODES)] = mat_ref[r, :]


def kernel(x, edge_index, W0, W1):
    src = edge_index[0].astype(jnp.int32)
    dst = edge_index[1].astype(jnp.int32)
    pad = E_PAD - N_EDGES
    src_p = jnp.concatenate([src, jnp.zeros((pad,), jnp.int32)]
                            ).reshape(NW, K_CHUNKS, CHUNK)
    dst_p = jnp.concatenate([dst, jnp.full((pad,), DUMP_ROW, jnp.int32)]
                            ).reshape(NW, K_CHUNKS, CHUNK)
    zero_h = jnp.zeros((ACC_N, HIDDEN), jnp.float32)
    zero_z = jnp.zeros((ACC_N, D_OUT), jnp.float32)

    h0 = pl.pallas_call(
        _mm0_body,
        out_shape=jax.ShapeDtypeStruct((N_NODES, HIDDEN), jnp.float32),
    )(x, W0)

    h_parts = _segsum_h(h0, src_p, dst_p, zero_h)

    z0 = pl.pallas_call(
        _mm1_body,
        grid=(1,),
        in_specs=[
            pl.BlockSpec((1, N_NODES, HIDDEN), lambda i: (0, 0, 0)),
            pl.BlockSpec((1, N_NODES, HIDDEN), lambda i: (1, 0, 0)),
            pl.BlockSpec((HIDDEN, D_OUT), lambda i: (0, 0)),
        ],
        out_specs=pl.BlockSpec((N_NODES, D_OUT), lambda i: (0, 0)),
        out_shape=jax.ShapeDtypeStruct((N_NODES, D_OUT), jnp.float32),
    )(h_parts, h_parts, W1)

    z_parts = _segsum_z(z0, src_p, dst_p, zero_z)

    grid = -(-N_NODES // _PRED_BM)
    z, pred = pl.pallas_call(
        _pred_body,
        grid=(grid,),
        in_specs=[
            pl.BlockSpec((1, ACC_N, D_OUT), lambda i: (0, 0, 0)),
            pl.BlockSpec((1, ACC_N, D_OUT), lambda i: (1, 0, 0)),
        ],
        out_specs=[
            pl.BlockSpec((_PRED_BM, D_OUT), lambda i: (i, 0)),
            pl.BlockSpec((_PRED_BM * N_NODES,), lambda i: (i,)),
        ],
        out_shape=[
            jax.ShapeDtypeStruct((N_NODES, D_OUT), jnp.float32),
            jax.ShapeDtypeStruct((N_NODES * N_NODES,), jnp.float32),
        ],
        scratch_shapes=[pltpu.VMEM((_PRED_BM, N_NODES), jnp.float32)],
    )(z_parts, z_parts)

    return z, pred
